# baseline (speedup 1.0000x reference)
import functools

import jax
import jax.numpy as jnp
from jax import lax
from jax.experimental import pallas as pl
from jax.experimental.pallas import tpu as pltpu
from jax.experimental.pallas import tpu_sc as plsc

NC, NS = 2, 16
NW = NC * NS
PADROWS = 2048
DEG = 5
D = 128


def _inv_body(idx_hbm, inv_hbm, idxv, invloc):
    nd = idx_hbm.shape[0]
    n_pad = inv_hbm.shape[0]
    per_w = n_pad // NW
    wid = lax.axis_index("s") * NC + lax.axis_index("c")
    lo = wid * per_w
    @pl.loop(0, per_w // 16)
    def _init(t):
        n = lo + t * 16 + lax.iota(jnp.int32, 16)
        invloc[pl.ds(t * 16, 16)] = nd + (n & (PADROWS - 1))

    ch = 2000

    @pl.loop(0, nd // ch)
    def _scan(c):
        pltpu.sync_copy(idx_hbm.at[pl.ds(c * ch, ch)], idxv)

        @pl.loop(0, ch // 16)
        def _vreg(t):
            v = idxv[pl.ds(t * 16, 16)]
            p = c * ch + t * 16 + lax.iota(jnp.int32, 16)
            m = (v >= lo) & (v < lo + per_w)
            plsc.store_scatter(invloc, [v - lo], p, mask=m)

    pltpu.sync_copy(invloc, inv_hbm.at[pl.ds(lo, per_w)])


def _sc_inv(idx, n_pad):
    mesh = plsc.VectorSubcoreMesh(core_axis_name="c", subcore_axis_name="s")
    f = pl.kernel(
        _inv_body,
        out_type=jax.ShapeDtypeStruct((n_pad,), jnp.int32),
        mesh=mesh,
        scratch_types=[
            pltpu.VMEM((2000,), jnp.int32),
            pltpu.VMEM((n_pad // NW,), jnp.int32),
        ],
        compiler_params=pltpu.CompilerParams(needs_layout_passes=False),
    )
    return f(idx)


_CHUNK = 112
NBUF = 2


def _gather_body(off, hext, inv, jkm, out, j_slot, src_slot,
                 rows0, rows1,
                 semi, semr0, semr1,
                 semo0, semo1):
    n_pad = inv.shape[0]
    n_part = out.shape[0]
    per_w = n_part // NW
    n_chunks = per_w // _CHUNK
    wid = lax.axis_index("s") * NC + lax.axis_index("c")
    loc = wid * per_w
    lo = off + loc
    rows = (rows0, rows1)
    semr = (semr0, semr1)
    semo = (semo0, semo1)

    for k in range(DEG):
        pltpu.sync_copy(jkm.at[pl.ds(k * n_pad + lo, per_w)], j_slot)

        @pl.loop(0, n_chunks)
        def _fire(cc):
            s = pl.ds(cc * _CHUNK, _CHUNK)
            pltpu.async_copy(inv.at[j_slot.at[s]], src_slot.at[s], semi)

        @pl.loop(0, n_chunks)
        def _drain(cc):
            s = pl.ds(cc * _CHUNK, _CHUNK)
            pltpu.make_async_copy(inv.at[j_slot.at[s]], src_slot.at[s],
                                  semi).wait()

        def g_start(cc, b):
            s = pl.ds(cc * _CHUNK, _CHUNK)
            pltpu.async_copy(hext.at[src_slot.at[s]], rows[b], semr[b])

        def g_wait(cc, b):
            s = pl.ds(cc * _CHUNK, _CHUNK)
            pltpu.make_async_copy(hext.at[src_slot.at[s]], rows[b],
                                  semr[b]).wait()

        def o_ref(cc):
            return out.at[pl.ds(loc + cc * _CHUNK, _CHUNK), pl.ds(k * D, D)]

        def o_start(cc, b):
            pltpu.async_copy(rows[b], o_ref(cc), semo[b])

        def o_wait(cc, b):
            pltpu.make_async_copy(rows[b], o_ref(cc), semo[b]).wait()

        @pl.loop(0, n_chunks // NBUF)
        def _pipe(t):
            for q in range(NBUF):
                cc = t * NBUF + q

                @pl.when(t > 0)
                def _():
                    o_wait(cc - NBUF, q)
                g_start(cc, q)
            for q in range(NBUF):
                cc = t * NBUF + q
                g_wait(cc, q)
                o_start(cc, q)

        for q in range(NBUF):
            o_wait(n_chunks - NBUF + q, q)

def _sc_gather(hext, inv, jkm, off, n_part):
    mesh = plsc.VectorSubcoreMesh(core_axis_name="c", subcore_axis_name="s")
    per_w = n_part // NW
    f = pl.kernel(
        functools.partial(_gather_body, off),
        out_type=jax.ShapeDtypeStruct((n_part, DEG * D), jnp.float32),
        mesh=mesh,
        scratch_types=[
            pltpu.VMEM((per_w,), jnp.int32),
            pltpu.VMEM((per_w,), jnp.int32),
            pltpu.VMEM((_CHUNK, D), jnp.float32),
            pltpu.VMEM((_CHUNK, D), jnp.float32),
            pltpu.SemaphoreType.DMA,
            pltpu.SemaphoreType.DMA,
            pltpu.SemaphoreType.DMA,
            pltpu.SemaphoreType.DMA,
            pltpu.SemaphoreType.DMA,
        ],
    )
    return f(hext, inv, jkm)


def _mlp_body(x_ref, w1_ref, b1_ref, w2_ref, b2_ref, o_ref):
    x16 = x_ref[...].astype(jnp.bfloat16)
    z = jnp.dot(x16, w1_ref[...], preferred_element_type=jnp.float32)
    z = jnp.maximum(z + b1_ref[...], 0.0).astype(jnp.bfloat16)
    y = jnp.dot(z, w2_ref[...], preferred_element_type=jnp.float32)
    o_ref[...] = y + b2_ref[...]


def _tc_mlp(x, W1t, b1, W2t, b2, n_rows, bn):
    grid = (-(-n_rows // bn),)
    return pl.pallas_call(
        _mlp_body,
        out_shape=jax.ShapeDtypeStruct((n_rows, D), jnp.float32),
        grid=grid,
        in_specs=[
            pl.BlockSpec((bn, DEG * D), lambda i: (i, 0)),
            pl.BlockSpec((DEG * D, D), lambda i: (0, 0)),
            pl.BlockSpec((1, D), lambda i: (0, 0)),
            pl.BlockSpec((D, D), lambda i: (0, 0)),
            pl.BlockSpec((1, D), lambda i: (0, 0)),
        ],
        out_specs=pl.BlockSpec((bn, D), lambda i: (i, 0)),
    )(x, W1t, b1, W2t, b2)


def kernel(h, h_above, idx, i, j, W1, b1, W2, b2):
    ND = h.shape[0]
    N = h_above.shape[0]
    n_pad = ((N + NW * 112 - 1) // (NW * 112)) * (NW * 112)

    inv = _sc_inv(idx.astype(jnp.int32), n_pad)

    hext = jnp.concatenate([h, jnp.zeros((PADROWS, D), h.dtype)], axis=0)
    jkm = jnp.zeros((DEG, n_pad), jnp.int32).at[:, :N].set(
        j.reshape(N, DEG).T.astype(jnp.int32)).reshape(-1)

    half = n_pad // 2
    bn = 448
    W1tb = W1.T.astype(jnp.bfloat16)
    W2tb = W2.T.astype(jnp.bfloat16)
    b1r, b2r = b1.reshape(1, D), b2.reshape(1, D)

    x0 = _sc_gather(hext, inv, jkm, 0, half)
    x1 = _sc_gather(hext, inv, jkm, half, n_pad - half)
    out0 = _tc_mlp(x0, W1tb, b1r, W2tb, b2r, half, bn)
    out1 = _tc_mlp(x1, W1tb, b1r, W2tb, b2r, N - half, bn)
    return jnp.concatenate([out0, out1], axis=0)


# speedup vs baseline: 1.1013x; 1.1013x over previous
import functools

import jax
import jax.numpy as jnp
from jax import lax
from jax.experimental import pallas as pl
from jax.experimental.pallas import tpu as pltpu
from jax.experimental.pallas import tpu_sc as plsc

NC, NS = 2, 16
NW = NC * NS
PADROWS = 2048
DEG = 5
D = 128


def _inv_body(idx_hbm, inv_hbm, idxv, invloc):
    nd = idx_hbm.shape[0]
    n_pad = inv_hbm.shape[0]
    per_w = n_pad // NW
    wid = lax.axis_index("s") * NC + lax.axis_index("c")
    lo = wid * per_w
    @pl.loop(0, per_w // 16)
    def _init(t):
        n = lo + t * 16 + lax.iota(jnp.int32, 16)
        invloc[pl.ds(t * 16, 16)] = nd + (n & (PADROWS - 1))

    ch = 2000

    @pl.loop(0, nd // ch)
    def _scan(c):
        pltpu.sync_copy(idx_hbm.at[pl.ds(c * ch, ch)], idxv)

        @pl.loop(0, ch // 16)
        def _vreg(t):
            v = idxv[pl.ds(t * 16, 16)]
            p = c * ch + t * 16 + lax.iota(jnp.int32, 16)
            m = (v >= lo) & (v < lo + per_w)
            plsc.store_scatter(invloc, [v - lo], p, mask=m)

    pltpu.sync_copy(invloc, inv_hbm.at[pl.ds(lo, per_w)])


def _sc_inv(idx, n_pad):
    mesh = plsc.VectorSubcoreMesh(core_axis_name="c", subcore_axis_name="s")
    f = pl.kernel(
        _inv_body,
        out_type=jax.ShapeDtypeStruct((n_pad,), jnp.int32),
        mesh=mesh,
        scratch_types=[
            pltpu.VMEM((2000,), jnp.int32),
            pltpu.VMEM((n_pad // NW,), jnp.int32),
        ],
        compiler_params=pltpu.CompilerParams(needs_layout_passes=False),
    )
    return f(idx)


_CHUNK = 112
NBUF = 4


def _gather_body(hext, inv, jkm, out, j_all, src_all,
                 rows0, rows1, rows2, rows3,
                 semi, semr0, semr1, semr2, semr3,
                 semo0, semo1, semo2, semo3):
    n_pad = inv.shape[0]
    per_w = n_pad // NW
    n_chunks = per_w // _CHUNK
    wid = lax.axis_index("s") * NC + lax.axis_index("c")
    lo = wid * per_w
    rows = (rows0, rows1, rows2, rows3)
    semr = (semr0, semr1, semr2, semr3)
    semo = (semo0, semo1, semo2, semo3)

    for k in range(DEG):
        pltpu.sync_copy(jkm.at[pl.ds(k * n_pad + lo, per_w)],
                        j_all.at[pl.ds(k * per_w, per_w)])

    def fire(k):
        @pl.loop(0, n_chunks)
        def _fire(cc):
            s = pl.ds(k * per_w + cc * _CHUNK, _CHUNK)
            pltpu.async_copy(inv.at[j_all.at[s]], src_all.at[s], semi)

    def drain(k):
        @pl.loop(0, n_chunks)
        def _drain(cc):
            s = pl.ds(k * per_w + cc * _CHUNK, _CHUNK)
            pltpu.make_async_copy(inv.at[j_all.at[s]], src_all.at[s],
                                  semi).wait()

    def pipeline(k):
        def g_start(cc, b):
            s = pl.ds(k * per_w + cc * _CHUNK, _CHUNK)
            pltpu.async_copy(hext.at[src_all.at[s]], rows[b], semr[b])

        def g_wait(cc, b):
            s = pl.ds(k * per_w + cc * _CHUNK, _CHUNK)
            pltpu.make_async_copy(hext.at[src_all.at[s]], rows[b],
                                  semr[b]).wait()

        def o_ref(cc):
            return out.at[pl.ds(lo + cc * _CHUNK, _CHUNK), pl.ds(k * D, D)]

        def o_start(cc, b):
            pltpu.async_copy(rows[b], o_ref(cc), semo[b])

        def o_wait(cc, b):
            pltpu.make_async_copy(rows[b], o_ref(cc), semo[b]).wait()

        @pl.loop(0, n_chunks // NBUF)
        def _pipe(t):
            for q in range(NBUF):
                cc = t * NBUF + q

                @pl.when(t > 0)
                def _():
                    o_wait(cc - NBUF, q)
                g_start(cc, q)
            for q in range(NBUF):
                cc = t * NBUF + q
                g_wait(cc, q)
                o_start(cc, q)

        for q in range(NBUF):
            o_wait(n_chunks - NBUF + q, q)

    fire(0)
    for k in range(DEG):
        drain(k)
        if k + 1 < DEG:
            fire(k + 1)
        pipeline(k)


def _sc_gather(hext, inv, jkm, n_pad):
    mesh = plsc.VectorSubcoreMesh(core_axis_name="c", subcore_axis_name="s")
    per_w = n_pad // NW
    f = pl.kernel(
        _gather_body,
        out_type=jax.ShapeDtypeStruct((n_pad, DEG * D), jnp.float32),
        mesh=mesh,
        scratch_types=[
            pltpu.VMEM((DEG * per_w,), jnp.int32),
            pltpu.VMEM((DEG * per_w,), jnp.int32),
            pltpu.VMEM((_CHUNK, D), jnp.float32),
            pltpu.VMEM((_CHUNK, D), jnp.float32),
            pltpu.VMEM((_CHUNK, D), jnp.float32),
            pltpu.VMEM((_CHUNK, D), jnp.float32),
            pltpu.SemaphoreType.DMA,
            pltpu.SemaphoreType.DMA,
            pltpu.SemaphoreType.DMA,
            pltpu.SemaphoreType.DMA,
            pltpu.SemaphoreType.DMA,
            pltpu.SemaphoreType.DMA,
            pltpu.SemaphoreType.DMA,
            pltpu.SemaphoreType.DMA,
            pltpu.SemaphoreType.DMA,
        ],
    )
    return f(hext, inv, jkm)


def _mlp_body(x_ref, w1_ref, b1_ref, w2_ref, b2_ref, o_ref):
    x16 = x_ref[...].astype(jnp.bfloat16)
    z = jnp.dot(x16, w1_ref[...], preferred_element_type=jnp.float32)
    z = jnp.maximum(z + b1_ref[...], 0.0).astype(jnp.bfloat16)
    y = jnp.dot(z, w2_ref[...], preferred_element_type=jnp.float32)
    o_ref[...] = y + b2_ref[...]


def _tc_mlp(x, W1t, b1, W2t, b2, n_rows, bn):
    grid = (n_rows // bn,)
    return pl.pallas_call(
        _mlp_body,
        out_shape=jax.ShapeDtypeStruct((n_rows, D), jnp.float32),
        grid=grid,
        in_specs=[
            pl.BlockSpec((bn, DEG * D), lambda i: (i, 0)),
            pl.BlockSpec((DEG * D, D), lambda i: (0, 0)),
            pl.BlockSpec((1, D), lambda i: (0, 0)),
            pl.BlockSpec((D, D), lambda i: (0, 0)),
            pl.BlockSpec((1, D), lambda i: (0, 0)),
        ],
        out_specs=pl.BlockSpec((bn, D), lambda i: (i, 0)),
    )(x, W1t, b1, W2t, b2)


def kernel(h, h_above, idx, i, j, W1, b1, W2, b2):
    ND = h.shape[0]
    N = h_above.shape[0]
    n_pad = ((N + NW * 112 - 1) // (NW * 112)) * (NW * 112)

    inv = _sc_inv(idx.astype(jnp.int32), n_pad)

    hext = jnp.concatenate([h, jnp.zeros((PADROWS, D), h.dtype)], axis=0)
    jkm = jnp.zeros((DEG, n_pad), jnp.int32).at[:, :N].set(
        j.reshape(N, DEG).T.astype(jnp.int32)).reshape(-1)

    x = _sc_gather(hext, inv, jkm, n_pad)

    out = _tc_mlp(x, W1.T.astype(jnp.bfloat16), b1.reshape(1, D),
                  W2.T.astype(jnp.bfloat16), b2.reshape(1, D), N, 1000)
    return out
